# baseline (speedup 1.0000x reference)
import jax
import jax.numpy as jnp
from jax.experimental import pallas as pl
from jax.experimental.pallas import tpu as pltpu

B, T, N, F = 128, 24, 4096, 2
R = 16
BB = 16
BP = 16
P = 10


def _pass1(x_ref, s_ref, c_ref):
    x = x_ref[...]
    fmask = jax.lax.broadcasted_iota(jnp.int32, (BB, T, F, N), 2) == 0
    valid = (x != -1.0) & fmask
    s_ref[...] = jnp.sum(jnp.where(valid, x, 0.0), axis=(1, 2))
    c_ref[...] = jnp.sum(valid.astype(jnp.float32), axis=(1, 2))


def _passG(s_ref, c_ref, cid_ref, gm_ref, reg_ref):
    s = s_ref[...]
    c = c_ref[...]
    gm = jnp.sum(s) / jnp.sum(c)
    gm_ref[0, 0] = gm
    cid = cid_ref[...]
    oh = (jax.lax.broadcasted_iota(jnp.int32, (R, N), 0) == cid
          ).astype(jnp.float32)
    dn = (((1,), (1,)), ((), ()))
    S = jax.lax.dot_general(oh, s, dn, preferred_element_type=jnp.float32)
    C = jax.lax.dot_general(oh, c, dn, preferred_element_type=jnp.float32)
    cnt = jnp.sum(oh, axis=1, keepdims=True)
    reg = (S + (T * cnt - C) * gm) / (T * cnt)
    reg_ref[...] = jnp.broadcast_to(reg[None, :, :], (P, R, B))


def _predK(gm_ref, s_ref, c_ref, pred_ref):
    gm = gm_ref[0, 0]
    mean = (s_ref[...] + (T - c_ref[...]) * gm) * (1.0 / T)
    pred_ref[...] = jnp.broadcast_to(mean[None, :, :], (P, BP, N))


def kernel(seq, cluster_id):
    seq_t = jnp.transpose(seq, (0, 1, 3, 2))
    cid_row = cluster_id.reshape(1, N).astype(jnp.int32)
    s, c = pl.pallas_call(
        _pass1,
        grid=(B // BB,),
        in_specs=[pl.BlockSpec((BB, T, F, N), lambda i: (i, 0, 0, 0))],
        out_specs=[pl.BlockSpec((BB, N), lambda i: (i, 0)),
                   pl.BlockSpec((BB, N), lambda i: (i, 0))],
        out_shape=[jax.ShapeDtypeStruct((B, N), jnp.float32),
                   jax.ShapeDtypeStruct((B, N), jnp.float32)],
    )(seq_t)
    gm, reg_t = pl.pallas_call(
        _passG,
        out_shape=[jax.ShapeDtypeStruct((1, 1), jnp.float32),
                   jax.ShapeDtypeStruct((P, R, B), jnp.float32)],
        out_specs=[pl.BlockSpec(memory_space=pltpu.SMEM),
                   pl.BlockSpec((P, R, B), lambda: (0, 0, 0))],
    )(s, c, cid_row)
    pred_t = pl.pallas_call(
        _predK,
        grid=(B // BP,),
        in_specs=[pl.BlockSpec(memory_space=pltpu.SMEM),
                  pl.BlockSpec((BP, N), lambda i: (i, 0)),
                  pl.BlockSpec((BP, N), lambda i: (i, 0))],
        out_specs=pl.BlockSpec((P, BP, N), lambda i: (0, i, 0)),
        out_shape=jax.ShapeDtypeStruct((P, B, N), jnp.float32),
    )(gm, s, c)
    pred = jnp.transpose(pred_t, (1, 0, 2))
    reg = jnp.transpose(reg_t, (2, 0, 1))
    return pred, reg


# speedup vs baseline: 1.0123x; 1.0123x over previous
import jax
import jax.numpy as jnp
from jax.experimental import pallas as pl
from jax.experimental.pallas import tpu as pltpu

B, T, N, F = 128, 24, 4096, 2
R = 16
BB = 8
BP = 16
P = 10


def _pass1(x_ref, s_ref, c_ref):
    x = x_ref[...]
    fmask = jax.lax.broadcasted_iota(jnp.int32, (BB, T, F, N), 2) == 0
    valid = (x != -1.0) & fmask
    s_ref[...] = jnp.sum(jnp.where(valid, x, 0.0), axis=(1, 2))
    c_ref[...] = jnp.sum(valid.astype(jnp.float32), axis=(1, 2))


def _passG(s_ref, c_ref, cid_ref, gm_ref, reg_ref):
    s = s_ref[...]
    c = c_ref[...]
    gm = jnp.sum(s) / jnp.sum(c)
    gm_ref[0, 0] = gm
    cid = cid_ref[...]
    oh = (jax.lax.broadcasted_iota(jnp.int32, (R, N), 0) == cid
          ).astype(jnp.float32)
    dn = (((1,), (1,)), ((), ()))
    S = jax.lax.dot_general(oh, s, dn, preferred_element_type=jnp.float32)
    C = jax.lax.dot_general(oh, c, dn, preferred_element_type=jnp.float32)
    cnt = jnp.sum(oh, axis=1, keepdims=True)
    reg = (S + (T * cnt - C) * gm) / (T * cnt)
    reg_ref[...] = jnp.broadcast_to(reg[None, :, :], (P, R, B))


def _predK(gm_ref, s_ref, c_ref, pred_ref):
    gm = gm_ref[0, 0]
    mean = (s_ref[...] + (T - c_ref[...]) * gm) * (1.0 / T)
    pred_ref[...] = jnp.broadcast_to(mean[None, :, :], (P, BP, N))


def kernel(seq, cluster_id):
    seq_t = jnp.transpose(seq, (0, 1, 3, 2))
    cid_row = cluster_id.reshape(1, N).astype(jnp.int32)
    s, c = pl.pallas_call(
        _pass1,
        grid=(B // BB,),
        in_specs=[pl.BlockSpec((BB, T, F, N), lambda i: (i, 0, 0, 0))],
        out_specs=[pl.BlockSpec((BB, N), lambda i: (i, 0)),
                   pl.BlockSpec((BB, N), lambda i: (i, 0))],
        out_shape=[jax.ShapeDtypeStruct((B, N), jnp.float32),
                   jax.ShapeDtypeStruct((B, N), jnp.float32)],
    )(seq_t)
    gm, reg_t = pl.pallas_call(
        _passG,
        out_shape=[jax.ShapeDtypeStruct((1, 1), jnp.float32),
                   jax.ShapeDtypeStruct((P, R, B), jnp.float32)],
        out_specs=[pl.BlockSpec(memory_space=pltpu.SMEM),
                   pl.BlockSpec((P, R, B), lambda: (0, 0, 0))],
    )(s, c, cid_row)
    pred_t = pl.pallas_call(
        _predK,
        grid=(B // BP,),
        in_specs=[pl.BlockSpec(memory_space=pltpu.SMEM),
                  pl.BlockSpec((BP, N), lambda i: (i, 0)),
                  pl.BlockSpec((BP, N), lambda i: (i, 0))],
        out_specs=pl.BlockSpec((P, BP, N), lambda i: (0, i, 0)),
        out_shape=jax.ShapeDtypeStruct((P, B, N), jnp.float32),
    )(gm, s, c)
    pred = jnp.transpose(pred_t, (1, 0, 2))
    reg = jnp.transpose(reg_t, (2, 0, 1))
    return pred, reg


# speedup vs baseline: 3.2176x; 3.1786x over previous
import jax
import jax.numpy as jnp
from jax.experimental import pallas as pl
from jax.experimental.pallas import tpu as pltpu

B, T, N, F = 128, 24, 4096, 2
R = 16
BB = 16
BP = 16
P = 10


def _pass1(x_ref, s_ref, c_ref):
    acc_r = jnp.zeros((BB, N), jnp.float32)
    acc_c = jnp.zeros((BB, N), jnp.float32)
    for t in range(T):
        xt = x_ref[:, t, 0, :]
        acc_r = acc_r + xt
        acc_c = acc_c + (xt != -1.0).astype(jnp.float32)
    s_ref[...] = acc_r + (T - acc_c)
    c_ref[...] = acc_c


def _passG(s_ref, c_ref, cid_ref, gm_ref, reg_ref):
    s = s_ref[...]
    c = c_ref[...]
    gm = jnp.sum(s) / jnp.sum(c)
    gm_ref[0, 0] = gm
    cid = cid_ref[...]
    oh = (jax.lax.broadcasted_iota(jnp.int32, (R, N), 0) == cid
          ).astype(jnp.float32)
    dn = (((1,), (1,)), ((), ()))
    S = jax.lax.dot_general(oh, s, dn, preferred_element_type=jnp.float32)
    C = jax.lax.dot_general(oh, c, dn, preferred_element_type=jnp.float32)
    cnt = jnp.sum(oh, axis=1, keepdims=True)
    reg = (S + (T * cnt - C) * gm) / (T * cnt)
    reg_ref[...] = jnp.broadcast_to(reg[None, :, :], (P, R, B))


def _predK(gm_ref, s_ref, c_ref, pred_ref):
    gm = gm_ref[0, 0]
    mean = (s_ref[...] + (T - c_ref[...]) * gm) * (1.0 / T)
    pred_ref[...] = jnp.broadcast_to(mean[None, :, :], (P, BP, N))


def kernel(seq, cluster_id):
    seq_t = jnp.transpose(seq, (0, 1, 3, 2))
    cid_row = cluster_id.reshape(1, N).astype(jnp.int32)
    s, c = pl.pallas_call(
        _pass1,
        grid=(B // BB,),
        in_specs=[pl.BlockSpec((BB, T, F, N), lambda i: (i, 0, 0, 0))],
        out_specs=[pl.BlockSpec((BB, N), lambda i: (i, 0)),
                   pl.BlockSpec((BB, N), lambda i: (i, 0))],
        out_shape=[jax.ShapeDtypeStruct((B, N), jnp.float32),
                   jax.ShapeDtypeStruct((B, N), jnp.float32)],
    )(seq_t)
    gm, reg_t = pl.pallas_call(
        _passG,
        out_shape=[jax.ShapeDtypeStruct((1, 1), jnp.float32),
                   jax.ShapeDtypeStruct((P, R, B), jnp.float32)],
        out_specs=[pl.BlockSpec(memory_space=pltpu.SMEM),
                   pl.BlockSpec((P, R, B), lambda: (0, 0, 0))],
    )(s, c, cid_row)
    pred_t = pl.pallas_call(
        _predK,
        grid=(B // BP,),
        in_specs=[pl.BlockSpec(memory_space=pltpu.SMEM),
                  pl.BlockSpec((BP, N), lambda i: (i, 0)),
                  pl.BlockSpec((BP, N), lambda i: (i, 0))],
        out_specs=pl.BlockSpec((P, BP, N), lambda i: (0, i, 0)),
        out_shape=jax.ShapeDtypeStruct((P, B, N), jnp.float32),
    )(gm, s, c)
    pred = jnp.transpose(pred_t, (1, 0, 2))
    reg = jnp.transpose(reg_t, (2, 0, 1))
    return pred, reg
